# Initial kernel scaffold; baseline (speedup 1.0000x reference)
import jax
import jax.numpy as jnp
from jax.experimental import pallas as pl


def kernel(x, edge_index, W1l, b1l, W1r, W2l, b2l, W2r, W3l, b3l, W3r, g1, be1, g2, be2):
    raise NotImplementedError("write your pallas kernel here")


# speedup vs baseline: 5.1631x; 5.1631x over previous
import functools

import jax
import jax.numpy as jnp
from jax import lax
from jax.experimental import pallas as pl
from jax.experimental.pallas import tpu as pltpu
from jax.experimental.pallas import tpu_sc as plsc

N = 10000
NP = 10240
E = 320000
D = 128

NC = 2
NS = 16
NW = NC * NS
EPW = E // NW
C = 80
NCHUNK = EPW // C
RPT = NP // NS
ZR = 32
WB = 128
CW = 8


def _make_sc_agg(with_cnt: bool):
    mesh = plsc.VectorSubcoreMesh(core_axis_name="c", subcore_axis_name="s")

    out_type = [jax.ShapeDtypeStruct((2 * NP, D), jnp.float32)]
    scratch = [
        pltpu.VMEM((ZR, D), jnp.float32),
        pltpu.VMEM((C,), jnp.int32),
        pltpu.VMEM((C,), jnp.int32),
        pltpu.VMEM((C, D), jnp.float32),
        pltpu.VMEM((WB, D), jnp.float32),
        pltpu.VMEM_SHARED((NP, D), jnp.float32),
        pltpu.SemaphoreType.DMA,
    ]
    if with_cnt:
        out_type.append(jax.ShapeDtypeStruct((2 * NP, CW), jnp.float32))
        scratch += [
            pltpu.VMEM((RPT, CW), jnp.float32),
            pltpu.VMEM((C, CW), jnp.float32),
            pltpu.VMEM_SHARED((NP, CW), jnp.float32),
        ]

    def body(h_hbm, src_hbm, dst_hbm, zA_hbm, zC_hbm, ones_hbm,
             out_agg, out_cnt, zA_v, src_v, dst_v, rows_v, st_v, acc_sh, sem,
             zC_v=None, ones_v=None, cnt_sh=None):
        cid = lax.axis_index("c")
        sid = lax.axis_index("s")
        w = cid * NS + sid
        rbase = sid * RPT

        pltpu.sync_copy(zA_hbm, zA_v)
        for j in range(RPT // ZR):
            pltpu.sync_copy(zA_v, acc_sh.at[pl.ds(rbase + j * ZR, ZR)])
        if with_cnt:
            pltpu.sync_copy(zC_hbm, zC_v)
            pltpu.sync_copy(ones_hbm, ones_v)
            pltpu.sync_copy(zC_v, cnt_sh.at[pl.ds(rbase, RPT)])
        plsc.subcore_barrier()

        def chunk(i, carry):
            base = w * EPW + i * C
            pltpu.sync_copy(src_hbm.at[pl.ds(base, C)], src_v)
            pltpu.sync_copy(dst_hbm.at[pl.ds(base, C)], dst_v)
            pltpu.async_copy(h_hbm.at[src_v], rows_v, sem).wait()
            pltpu.sync_copy(rows_v, acc_sh.at[dst_v], add=True)
            if with_cnt:
                pltpu.sync_copy(ones_v, cnt_sh.at[dst_v], add=True)
            return carry

        lax.fori_loop(0, NCHUNK, chunk, 0)
        plsc.subcore_barrier()

        obase = cid * NP + rbase
        for j in range(RPT // WB):
            pltpu.sync_copy(acc_sh.at[pl.ds(rbase + j * WB, WB)], st_v)
            pltpu.sync_copy(st_v, out_agg.at[pl.ds(obase + j * WB, WB)])
        if with_cnt:
            pltpu.sync_copy(cnt_sh.at[pl.ds(rbase, RPT)], zC_v)
            pltpu.sync_copy(zC_v, out_cnt.at[pl.ds(obase, RPT)])

    params = pltpu.CompilerParams(use_tc_tiling_on_sc=False)
    if with_cnt:
        def body_cnt(h, s, d, zA, zC, on, oa, oc, *scr):
            body(h, s, d, zA, zC, on, oa, oc, *scr[:7],
                 zC_v=scr[7], ones_v=scr[8], cnt_sh=scr[9])
        return pl.kernel(body_cnt, mesh=mesh, out_type=out_type,
                         scratch_types=scratch, compiler_params=params)
    else:
        def body_nc(h, s, d, zA, oa, *scr):
            body(h, s, d, zA, None, None, oa, None, *scr)
        return pl.kernel(body_nc, mesh=mesh, out_type=out_type,
                         scratch_types=scratch, compiler_params=params)


_sc_agg_cnt = _make_sc_agg(True)
_sc_agg = _make_sc_agg(False)


def _tc_body(p0_ref, p1_ref, c0_ref, c1_ref, h_ref, wl_ref, bl_ref, wr_ref,
             g_ref, be_ref, o_ref, *, mode):
    cnt = jnp.maximum(c0_ref[:, 0:1] + c1_ref[:, 0:1], 1.0)
    mean = (p0_ref[...] + p1_ref[...]) / cnt
    y = jnp.dot(mean, wl_ref[...], preferred_element_type=jnp.float32)
    y = y + bl_ref[...]
    y = y + jnp.dot(h_ref[...], wr_ref[...], preferred_element_type=jnp.float32)
    if mode == 0:
        y = jnp.maximum(y * g_ref[...] + be_ref[...], 0.0)
    else:
        m = jnp.max(y, axis=1, keepdims=True)
        y = y - m
        y = y - jnp.log(jnp.sum(jnp.exp(y), axis=1, keepdims=True))
    o_ref[...] = y


_TB = 2000


def _tc_layer(p0, p1, c0, c1, h, Wl, bl, Wr, g, be, mode):
    grid = (N // _TB,)
    row = lambda i: (i, 0)
    rep = lambda i: (0, 0)
    return pl.pallas_call(
        functools.partial(_tc_body, mode=mode),
        grid=grid,
        in_specs=[
            pl.BlockSpec((_TB, D), row),
            pl.BlockSpec((_TB, D), row),
            pl.BlockSpec((_TB, CW), row),
            pl.BlockSpec((_TB, CW), row),
            pl.BlockSpec((_TB, D), row),
            pl.BlockSpec((D, D), rep),
            pl.BlockSpec((1, D), rep),
            pl.BlockSpec((D, D), rep),
            pl.BlockSpec((1, D), rep),
            pl.BlockSpec((1, D), rep),
        ],
        out_specs=pl.BlockSpec((_TB, D), row),
        out_shape=jax.ShapeDtypeStruct((N, D), jnp.float32),
    )(p0, p1, c0, c1, h, Wl, bl.reshape(1, D), Wr,
      g.reshape(1, D), be.reshape(1, D))


def kernel(x, edge_index, W1l, b1l, W1r, W2l, b2l, W2r, W3l, b3l, W3r,
           g1, be1, g2, be2):
    src = edge_index[0]
    dst = edge_index[1]
    zA = jnp.zeros((ZR, D), jnp.float32)
    zC = jnp.zeros((RPT, CW), jnp.float32)
    onesC = jnp.ones((C, CW), jnp.float32)

    agg, cnt = _sc_agg_cnt(x, src, dst, zA, zC, onesC)
    c0, c1 = cnt[:N], cnt[NP:NP + N]
    h = _tc_layer(agg[:N], agg[NP:NP + N], c0, c1, x, W1l, b1l, W1r, g1, be1, 0)
    (agg,) = _sc_agg(h, src, dst, zA)
    h = _tc_layer(agg[:N], agg[NP:NP + N], c0, c1, h, W2l, b2l, W2r, g2, be2, 0)
    (agg,) = _sc_agg(h, src, dst, zA)
    return _tc_layer(agg[:N], agg[NP:NP + N], c0, c1, h, W3l, b3l, W3r, g1, be1, 1)
